# baseline (speedup 1.0000x reference)
import functools

import jax
import jax.numpy as jnp
from jax import lax
from jax.experimental import pallas as pl
from jax.experimental.pallas import tpu as pltpu
from jax.experimental.pallas import tpu_sc as plsc

N = 10000
E = 320000
D = 128
G = 64

NC = 2
NS = 16
NW = NC * NS
C = 125
K = 80
KH = 16
NSB = K // KH
N_PAD = 10240
ZROWS = N_PAD // NS

_sc_mesh = plsc.VectorSubcoreMesh(core_axis_name="c", subcore_axis_name="s")


@functools.partial(
    pl.kernel,
    mesh=_sc_mesh,
    out_type=jax.ShapeDtypeStruct((NC, N_PAD, D), jnp.float32),
    scratch_types=[
        pltpu.VMEM((KH, C), jnp.int32),
        pltpu.VMEM((KH, C), jnp.int32),
        pltpu.VMEM((KH, C), jnp.int32),
        pltpu.VMEM((KH, C), jnp.int32),
        pltpu.VMEM((C, D), jnp.float32),
        pltpu.VMEM((C, D), jnp.float32),
        pltpu.VMEM_SHARED((N_PAD, D), jnp.float32),
        pltpu.SemaphoreType.DMA,
        pltpu.SemaphoreType.DMA,
        pltpu.SemaphoreType.DMA,
        pltpu.SemaphoreType.DMA,
        pltpu.SemaphoreType.DMA,
        pltpu.SemaphoreType.DMA,
        pltpu.SemaphoreType.DMA,
    ],
)
def _sc_agg(x_hbm, e_hbm, zeros_hbm, out_hbm,
            ss0, ss1, ds0, ds1, g0, g1, acc,
            zsem, ssem_a, ssem_b, dsem_a, dsem_b, gsem_a, gsem_b):
    cid = lax.axis_index("c")
    sid = lax.axis_index("s")
    wid = sid * NC + cid
    sss = (ss0, ss1)
    dss = (ds0, ds1)
    ssems = (ssem_a, ssem_b)
    dsems = (dsem_a, dsem_b)

    def slab_start(s):
        pltpu.async_copy(e_hbm.at[0, wid, pl.ds(s * KH, KH)],
                         sss[s % 2], ssems[s % 2])
        pltpu.async_copy(e_hbm.at[1, wid, pl.ds(s * KH, KH)],
                         dss[s % 2], dsems[s % 2])

    def slab_wait(s):
        pltpu.make_async_copy(e_hbm.at[0, wid, pl.ds(s * KH, KH)],
                              sss[s % 2], ssems[s % 2]).wait()
        pltpu.make_async_copy(e_hbm.at[1, wid, pl.ds(s * KH, KH)],
                              dss[s % 2], dsems[s % 2]).wait()

    def gather_start(ss, c, buf, sem):
        pltpu.async_copy(x_hbm.at[ss.at[c]], buf, sem)

    def gather_wait(ss, c, buf, sem):
        pltpu.make_async_copy(x_hbm.at[ss.at[c]], buf, sem).wait()

    def scatter_add(ds_, c, buf):
        pltpu.sync_copy(buf, acc.at[ds_.at[c]], add=True)

    slab_start(0)
    slab_start(1)
    z0 = sid * ZROWS
    for i in range(ZROWS // 128):
        pltpu.async_copy(zeros_hbm, acc.at[pl.ds(z0 + i * 128, 128)], zsem)
    for i in range(ZROWS // 128):
        pltpu.make_async_copy(
            zeros_hbm, acc.at[pl.ds(z0 + i * 128, 128)], zsem).wait()
    plsc.subcore_barrier()
    slab_wait(0)
    gather_start(ss0, 0, g0, gsem_a)
    gather_start(ss0, 1, g1, gsem_b)

    for s in range(NSB):
        ss = sss[s % 2]
        ds_ = dss[s % 2]
        nxt = sss[(s + 1) % 2]

        def body(j, carry):
            c0 = 2 * j
            gather_wait(ss, c0, g0, gsem_a)
            scatter_add(ds_, c0, g0)
            gather_start(ss, c0 + 2, g0, gsem_a)
            gather_wait(ss, c0 + 1, g1, gsem_b)
            scatter_add(ds_, c0 + 1, g1)
            gather_start(ss, c0 + 3, g1, gsem_b)
            return carry

        lax.fori_loop(0, KH // 2 - 1, body, 0)

        if s + 1 < NSB:
            slab_wait(s + 1)
            gather_wait(ss, KH - 2, g0, gsem_a)
            scatter_add(ds_, KH - 2, g0)
            gather_start(nxt, 0, g0, gsem_a)
            gather_wait(ss, KH - 1, g1, gsem_b)
            scatter_add(ds_, KH - 1, g1)
            gather_start(nxt, 1, g1, gsem_b)
            if s + 2 < NSB:
                slab_start(s + 2)
        else:
            gather_wait(ss, KH - 2, g0, gsem_a)
            scatter_add(ds_, KH - 2, g0)
            gather_wait(ss, KH - 1, g1, gsem_b)
            scatter_add(ds_, KH - 1, g1)

    plsc.subcore_barrier()
    pltpu.sync_copy(acc.at[pl.ds(z0, ZROWS)], out_hbm.at[cid, pl.ds(z0, ZROWS)])


BLK = 2000
NB = N // BLK


def _tc_body(x_ref, p_ref, b_ref,
             w1_ref, b1_ref, w2_ref, b2_ref, w3_ref, b3_ref, out_ref):
    f32 = jnp.float32
    h = x_ref[...] + p_ref[0] + p_ref[1]
    h = jnp.maximum(jnp.dot(h, w1_ref[...], preferred_element_type=f32)
                    + b1_ref[...], 0.0)
    h = jnp.maximum(jnp.dot(h, w2_ref[...], preferred_element_type=f32)
                    + b2_ref[...], 0.0)
    o = jnp.dot(h, w3_ref[...], preferred_element_type=f32) + b3_ref[...]
    seg = b_ref[0]
    gids = lax.broadcasted_iota(jnp.int32, (G, BLK), 0)
    onehot = (seg == gids).astype(f32)
    acc = jnp.dot(onehot, o, preferred_element_type=f32)

    @pl.when(pl.program_id(0) == 0)
    def _():
        out_ref[...] = acc

    @pl.when(pl.program_id(0) != 0)
    def _():
        out_ref[...] += acc


_tc_mlp_pool = pl.pallas_call(
    _tc_body,
    grid=(NB,),
    in_specs=[
        pl.BlockSpec((BLK, D), lambda i: (i, 0)),
        pl.BlockSpec((2, BLK, D), lambda i: (0, i, 0)),
        pl.BlockSpec((1, 1, BLK), lambda i: (i, 0, 0)),
        pl.BlockSpec((D, D), lambda i: (0, 0)),
        pl.BlockSpec((1, D), lambda i: (0, 0)),
        pl.BlockSpec((D, D), lambda i: (0, 0)),
        pl.BlockSpec((1, D), lambda i: (0, 0)),
        pl.BlockSpec((D, D), lambda i: (0, 0)),
        pl.BlockSpec((1, D), lambda i: (0, 0)),
    ],
    out_specs=pl.BlockSpec((G, D), lambda i: (0, 0)),
    out_shape=jax.ShapeDtypeStruct((G, D), jnp.float32),
    compiler_params=pltpu.CompilerParams(
        dimension_semantics=("arbitrary",)),
)


def kernel(x, edge_index, batch, W1, b1, W2, b2, W3, b3):
    e3 = edge_index.reshape(2, NW, K, C)
    zeros = jnp.zeros((128, D), jnp.float32)
    partials = _sc_agg(x, e3, zeros)
    pooled = _tc_mlp_pool(
        x, partials, batch.reshape(NB, 1, BLK),
        W1, b1.reshape(1, D), W2, b2.reshape(1, D), W3, b3.reshape(1, D))
    return pooled


# speedup vs baseline: 1.0709x; 1.0709x over previous
import functools

import jax
import jax.numpy as jnp
from jax import lax
from jax.experimental import pallas as pl
from jax.experimental.pallas import tpu as pltpu
from jax.experimental.pallas import tpu_sc as plsc

N = 10000
E = 320000
D = 128
G = 64

NC = 2
NS = 16
NW = NC * NS
C = 125
K = 80
KH = 16
NSB = K // KH
N_PAD = 10240
ZROWS = N_PAD // NS

_sc_mesh = plsc.VectorSubcoreMesh(core_axis_name="c", subcore_axis_name="s")


@functools.partial(
    pl.kernel,
    mesh=_sc_mesh,
    out_type=jax.ShapeDtypeStruct((NC, N_PAD, D), jnp.float32),
    scratch_types=[
        pltpu.VMEM((KH, C), jnp.int32),
        pltpu.VMEM((KH, C), jnp.int32),
        pltpu.VMEM((KH, C), jnp.int32),
        pltpu.VMEM((KH, C), jnp.int32),
        pltpu.VMEM((C, D), jnp.float32),
        pltpu.VMEM((C, D), jnp.float32),
        pltpu.VMEM_SHARED((N_PAD, D), jnp.float32),
        pltpu.SemaphoreType.DMA,
        pltpu.SemaphoreType.DMA,
        pltpu.SemaphoreType.DMA,
        pltpu.SemaphoreType.DMA,
        pltpu.SemaphoreType.DMA,
        pltpu.SemaphoreType.DMA,
    ],
)
def _sc_agg(x_hbm, e_hbm, zeros_hbm, out_hbm,
            ss0, ss1, ds0, ds1, g0, g1, acc,
            ssem_a, ssem_b, dsem_a, dsem_b, gsem_a, gsem_b):
    cid = lax.axis_index("c")
    sid = lax.axis_index("s")
    wid = sid * NC + cid
    sss = (ss0, ss1)
    dss = (ds0, ds1)
    ssems = (ssem_a, ssem_b)
    dsems = (dsem_a, dsem_b)

    def slab_start(s):
        pltpu.async_copy(e_hbm.at[0, wid, pl.ds(s * KH, KH)],
                         sss[s % 2], ssems[s % 2])
        pltpu.async_copy(e_hbm.at[1, wid, pl.ds(s * KH, KH)],
                         dss[s % 2], dsems[s % 2])

    def slab_wait(s):
        pltpu.make_async_copy(e_hbm.at[0, wid, pl.ds(s * KH, KH)],
                              sss[s % 2], ssems[s % 2]).wait()
        pltpu.make_async_copy(e_hbm.at[1, wid, pl.ds(s * KH, KH)],
                              dss[s % 2], dsems[s % 2]).wait()

    def gather_start(ss, c, buf, sem):
        pltpu.async_copy(x_hbm.at[ss.at[c]], buf, sem)

    def gather_wait(ss, c, buf, sem):
        pltpu.make_async_copy(x_hbm.at[ss.at[c]], buf, sem).wait()

    def scatter_add(ds_, c, buf):
        pltpu.sync_copy(buf, acc.at[ds_.at[c]], add=True)

    slab_start(0)
    slab_start(1)
    z0 = sid * ZROWS
    pltpu.sync_copy(zeros_hbm.at[pl.ds(z0, ZROWS)], acc.at[pl.ds(z0, ZROWS)])
    plsc.subcore_barrier()
    slab_wait(0)
    gather_start(ss0, 0, g0, gsem_a)
    gather_start(ss0, 1, g1, gsem_b)

    for s in range(NSB):
        ss = sss[s % 2]
        ds_ = dss[s % 2]
        nxt = sss[(s + 1) % 2]

        def body(j, carry):
            c0 = 2 * j
            gather_wait(ss, c0, g0, gsem_a)
            scatter_add(ds_, c0, g0)
            gather_start(ss, c0 + 2, g0, gsem_a)
            gather_wait(ss, c0 + 1, g1, gsem_b)
            scatter_add(ds_, c0 + 1, g1)
            gather_start(ss, c0 + 3, g1, gsem_b)
            return carry

        lax.fori_loop(0, KH // 2 - 1, body, 0)

        if s + 1 < NSB:
            slab_wait(s + 1)
            gather_wait(ss, KH - 2, g0, gsem_a)
            scatter_add(ds_, KH - 2, g0)
            gather_start(nxt, 0, g0, gsem_a)
            gather_wait(ss, KH - 1, g1, gsem_b)
            scatter_add(ds_, KH - 1, g1)
            gather_start(nxt, 1, g1, gsem_b)
            if s + 2 < NSB:
                slab_start(s + 2)
        else:
            gather_wait(ss, KH - 2, g0, gsem_a)
            scatter_add(ds_, KH - 2, g0)
            gather_wait(ss, KH - 1, g1, gsem_b)
            scatter_add(ds_, KH - 1, g1)

    plsc.subcore_barrier()
    pltpu.sync_copy(acc.at[pl.ds(z0, ZROWS)], out_hbm.at[cid, pl.ds(z0, ZROWS)])


BLK = 2000
NB = N // BLK


def _tc_body(x_ref, p_ref, b_ref,
             w1_ref, b1_ref, w2_ref, b2_ref, w3_ref, b3_ref, out_ref):
    f32 = jnp.float32
    h = x_ref[...] + p_ref[0] + p_ref[1]
    h = jnp.maximum(jnp.dot(h, w1_ref[...], preferred_element_type=f32)
                    + b1_ref[...], 0.0)
    h = jnp.maximum(jnp.dot(h, w2_ref[...], preferred_element_type=f32)
                    + b2_ref[...], 0.0)
    o = jnp.dot(h, w3_ref[...], preferred_element_type=f32) + b3_ref[...]
    seg = b_ref[0]
    gids = lax.broadcasted_iota(jnp.int32, (G, BLK), 0)
    onehot = (seg == gids).astype(f32)
    acc = jnp.dot(onehot, o, preferred_element_type=f32)

    @pl.when(pl.program_id(0) == 0)
    def _():
        out_ref[...] = acc

    @pl.when(pl.program_id(0) != 0)
    def _():
        out_ref[...] += acc


_tc_mlp_pool = pl.pallas_call(
    _tc_body,
    grid=(NB,),
    in_specs=[
        pl.BlockSpec((BLK, D), lambda i: (i, 0)),
        pl.BlockSpec((2, BLK, D), lambda i: (0, i, 0)),
        pl.BlockSpec((1, 1, BLK), lambda i: (i, 0, 0)),
        pl.BlockSpec((D, D), lambda i: (0, 0)),
        pl.BlockSpec((1, D), lambda i: (0, 0)),
        pl.BlockSpec((D, D), lambda i: (0, 0)),
        pl.BlockSpec((1, D), lambda i: (0, 0)),
        pl.BlockSpec((D, D), lambda i: (0, 0)),
        pl.BlockSpec((1, D), lambda i: (0, 0)),
    ],
    out_specs=pl.BlockSpec((G, D), lambda i: (0, 0)),
    out_shape=jax.ShapeDtypeStruct((G, D), jnp.float32),
    compiler_params=pltpu.CompilerParams(
        dimension_semantics=("arbitrary",)),
)


def kernel(x, edge_index, batch, W1, b1, W2, b2, W3, b3):
    e3 = edge_index.reshape(2, NW, K, C)
    zeros = jnp.zeros((N_PAD, D), jnp.float32)
    partials = _sc_agg(x, e3, zeros)
    pooled = _tc_mlp_pool(
        x, partials, batch.reshape(NB, 1, BLK),
        W1, b1.reshape(1, D), W2, b2.reshape(1, D), W3, b3.reshape(1, D))
    return pooled
